# baseline (speedup 1.0000x reference)
import functools

import jax
import jax.numpy as jnp
from jax import lax
from jax.experimental import pallas as pl
from jax.experimental.pallas import tpu as pltpu
from jax.experimental.pallas import tpu_sc as plsc

_TABLE_ROWS = 3375
_H = 16
_N = 512
_N2 = _N * _N
_NW = 32
_PER_W = _N2 // _NW
_CHUNK = 2048
_NCHUNK = _PER_W // _CHUNK


def _bias_body(table_hbm, idx_hbm, out_hbm, table_v, idx_v, outT_v,
               idx_sem, out_sem):
    wid = lax.axis_index("s") * 2 + lax.axis_index("c")
    base = wid * _PER_W
    mrow0 = wid * (_PER_W // _N)

    def start_idx(c):
        b = lax.rem(c, 2)
        pltpu.async_copy(idx_hbm.at[pl.ds(mrow0 + c * (_CHUNK // _N),
                                          _CHUNK // _N), :],
                         idx_v.at[b], idx_sem.at[b])

    def wait_idx(c):
        b = lax.rem(c, 2)
        pltpu.make_async_copy(idx_hbm.at[pl.ds(0, _CHUNK // _N), :],
                              idx_v.at[b], idx_sem.at[b]).wait()

    def out_dst(c):
        return out_hbm.at[0, :, pl.ds(mrow0 + c * (_CHUNK // _N),
                                      _CHUNK // _N), :]

    def start_out(c):
        b = lax.rem(c, 2)
        pltpu.async_copy(outT_v.at[b], out_dst(c), out_sem.at[b])

    def wait_out(c):
        b = lax.rem(c, 2)
        pltpu.make_async_copy(outT_v.at[b], out_dst(c), out_sem.at[b]).wait()

    start_idx(0)
    start_idx(1)
    pltpu.sync_copy(table_hbm, table_v)

    def chunk_body(c, carry):
        b = lax.rem(c, 2)
        wait_idx(c)

        @pl.when(c >= 2)
        def _():
            wait_out(c - 2)

        @plsc.parallel_loop(0, _N // 16, unroll=2)
        def _gather(g):
            for r in range(_CHUNK // _N):
                vidx = idx_v[b, r, pl.ds(g * 16, 16)]
                for h in range(_H):
                    v = plsc.load_gather(table_v,
                                         [vidx + h * _TABLE_ROWS])
                    outT_v[b, h, r, pl.ds(g * 16, 16)] = v

        start_out(c)

        @pl.when(c + 2 < _NCHUNK)
        def _():
            start_idx(c + 2)

        return carry

    lax.fori_loop(0, _NCHUNK, chunk_body, 0)
    wait_out(_NCHUNK - 2)
    wait_out(_NCHUNK - 1)


@functools.partial(
    pl.kernel,
    mesh=plsc.VectorSubcoreMesh(core_axis_name="c", subcore_axis_name="s"),
    compiler_params=pltpu.CompilerParams(needs_layout_passes=False),
    out_type=jax.ShapeDtypeStruct((1, _H, _N, _N), jnp.float32),
    scratch_types=[
        pltpu.VMEM((_TABLE_ROWS * _H,), jnp.float32),
        pltpu.VMEM((2, _CHUNK // _N, _N), jnp.int32),
        pltpu.VMEM((2, _H, _CHUNK // _N, _N), jnp.float32),
        pltpu.SemaphoreType.DMA((2,)),
        pltpu.SemaphoreType.DMA((2,)),
    ],
)
def _bias_call(table_hbm, idx_hbm, out_hbm, table_v, idx_v, outT_v,
               idx_sem, out_sem):
    _bias_body(table_hbm, idx_hbm, out_hbm, table_v, idx_v, outT_v,
               idx_sem, out_sem)


def kernel(relative_position_bias_table, relative_position_index):
    table_flat = relative_position_bias_table.T.reshape(-1)
    return _bias_call(table_flat, relative_position_index)


# speedup vs baseline: 1.2906x; 1.2906x over previous
import functools

import jax
import jax.numpy as jnp
from jax import lax
from jax.experimental import pallas as pl
from jax.experimental.pallas import tpu as pltpu
from jax.experimental.pallas import tpu_sc as plsc

_TABLE_ROWS = 3375
_H = 16
_N = 512
_N2 = _N * _N
_NW = 32
_PER_W = _N2 // _NW
_CHUNK = 1024
_NCHUNK = _PER_W // _CHUNK


def _bias_body(table_hbm, idx_hbm, out_hbm, table_v, idx_v, outT_v,
               idx_sem, out_sem):
    wid = lax.axis_index("s") * 2 + lax.axis_index("c")
    base = wid * _PER_W
    mrow0 = wid * (_PER_W // _N)

    def start_idx(c):
        b = lax.rem(c, 2)
        pltpu.async_copy(idx_hbm.at[pl.ds(mrow0 + c * (_CHUNK // _N),
                                          _CHUNK // _N), :],
                         idx_v.at[b], idx_sem.at[b])

    def wait_idx(c):
        b = lax.rem(c, 2)
        pltpu.make_async_copy(idx_hbm.at[pl.ds(0, _CHUNK // _N), :],
                              idx_v.at[b], idx_sem.at[b]).wait()

    def out_dst(c):
        return out_hbm.at[0, :, pl.ds(mrow0 + c * (_CHUNK // _N),
                                      _CHUNK // _N), :]

    def start_out(c):
        b = lax.rem(c, 2)
        pltpu.async_copy(outT_v.at[b], out_dst(c), out_sem.at[b])

    def wait_out(c):
        b = lax.rem(c, 2)
        pltpu.make_async_copy(outT_v.at[b], out_dst(c), out_sem.at[b]).wait()

    start_idx(0)
    start_idx(1)
    pltpu.sync_copy(table_hbm, table_v)

    def chunk_body(c, carry):
        b = lax.rem(c, 2)
        wait_idx(c)

        @pl.when(c >= 2)
        def _():
            wait_out(c - 2)

        @plsc.parallel_loop(0, _N // 16)
        def _gather(g):
            for r in range(_CHUNK // _N):
                vidx = idx_v[b, r, pl.ds(g * 16, 16)]
                for h in range(_H):
                    v = plsc.load_gather(table_v,
                                         [vidx + h * _TABLE_ROWS])
                    outT_v[b, h, r, pl.ds(g * 16, 16)] = v

        start_out(c)

        @pl.when(c + 2 < _NCHUNK)
        def _():
            start_idx(c + 2)

        return carry

    lax.fori_loop(0, _NCHUNK, chunk_body, 0)
    wait_out(_NCHUNK - 2)
    wait_out(_NCHUNK - 1)


@functools.partial(
    pl.kernel,
    mesh=plsc.VectorSubcoreMesh(core_axis_name="c", subcore_axis_name="s"),
    compiler_params=pltpu.CompilerParams(needs_layout_passes=False),
    out_type=jax.ShapeDtypeStruct((1, _H, _N, _N), jnp.float32),
    scratch_types=[
        pltpu.VMEM((_TABLE_ROWS * _H,), jnp.float32),
        pltpu.VMEM((2, _CHUNK // _N, _N), jnp.int32),
        pltpu.VMEM((2, _H, _CHUNK // _N, _N), jnp.float32),
        pltpu.SemaphoreType.DMA((2,)),
        pltpu.SemaphoreType.DMA((2,)),
    ],
)
def _bias_call(table_hbm, idx_hbm, out_hbm, table_v, idx_v, outT_v,
               idx_sem, out_sem):
    _bias_body(table_hbm, idx_hbm, out_hbm, table_v, idx_v, outT_v,
               idx_sem, out_sem)


def kernel(relative_position_bias_table, relative_position_index):
    table_flat = relative_position_bias_table.T.reshape(-1)
    return _bias_call(table_flat, relative_position_index)
